# baseline (speedup 1.0000x reference)
import functools

import jax
import jax.numpy as jnp
from jax import lax
from jax.experimental import pallas as pl
from jax.experimental.pallas import tpu as pltpu
from jax.experimental.pallas import tpu_sc as plsc

_W = 128
_NCHUNK = 2
_BB = 8


def _sc_gather(table, idx):
    n = idx.shape[1]
    e = table.shape[1]
    mesh = plsc.VectorSubcoreMesh(core_axis_name="core", subcore_axis_name="subcore")

    @functools.partial(
        pl.kernel,
        out_type=jax.ShapeDtypeStruct((n, e), table.dtype),
        mesh=mesh,
    )
    def gather_kernel(table_hbm, idx_hbm, out_hbm):
        def body(idx_vmem, out_vmem):
            pltpu.sync_copy(table_hbm.at[idx_vmem.at[0]], out_vmem)

        pltpu.emit_pipeline(
            body,
            grid=(n // _W,),
            in_specs=[pl.BlockSpec((1, _W), index_map=lambda i: (0, i))],
            out_specs=[pl.BlockSpec((_W, e), index_map=lambda i: (i, 0))],
            core_axis_name=("core", "subcore"),
            dimension_semantics=(pltpu.PARALLEL,),
        )(idx_hbm, out_hbm)

    return gather_kernel(table, idx)


def _ln_math(ex_ref, segf_ref, ep_ref, segw_ref, gamma_ref, beta_ref, out_ref):
    ex = ex_ref[...].astype(jnp.bfloat16)
    segf = segf_ref[...]
    ep = ep_ref[...] + segw_ref[0]
    ds = segw_ref[1] - segw_ref[0]
    h = ex + ep[None, :, :] + segf[:, :, None] * ds[None, None, :]
    mean = jnp.mean(h, axis=-1, keepdims=True)
    d = h - mean
    var = jnp.mean(d * d, axis=-1, keepdims=True)
    inv = lax.rsqrt(var.astype(jnp.float32) + 1e-5).astype(jnp.bfloat16)
    y = d * inv * gamma_ref[0][None, None, :] + beta_ref[0][None, None, :]
    out_ref[...] = y.astype(jnp.float32)


def _ln_body_first(ex_ref, segf_ref, ep_ref, segw_ref, gamma_ref, beta_ref, out_ref):
    _ln_math(ex_ref, segf_ref, ep_ref, segw_ref, gamma_ref, beta_ref, out_ref)


def _ln_body_chain(prev_ref, ex_ref, segf_ref, ep_ref, segw_ref, gamma_ref,
                   beta_ref, out_ref):
    del prev_ref
    _ln_math(ex_ref, segf_ref, ep_ref, segw_ref, gamma_ref, beta_ref, out_ref)


def _tc_chunk(prev, ex_c, segf, ep, segw, gamma2, beta2, c, nblk):
    b, l = segf.shape
    e = ex_c.shape[-1]
    off = c * nblk
    data_specs = [
        pl.BlockSpec((_BB, l, e), lambda i: (i, 0, 0)),
        pl.BlockSpec((_BB, l), lambda i, off=off: (off + i, 0)),
        pl.BlockSpec((l, e), lambda i: (0, 0)),
        pl.BlockSpec((2, e), lambda i: (0, 0)),
        pl.BlockSpec((1, e), lambda i: (0, 0)),
        pl.BlockSpec((1, e), lambda i: (0, 0)),
    ]
    out_spec = pl.BlockSpec((_BB, l, e), lambda i, off=off: (off + i, 0, 0))
    out_shape = jax.ShapeDtypeStruct((b, l, e), jnp.float32)
    if prev is None:
        return pl.pallas_call(
            _ln_body_first,
            grid=(nblk,),
            in_specs=data_specs,
            out_specs=out_spec,
            out_shape=out_shape,
        )(ex_c, segf, ep, segw, gamma2, beta2)
    return pl.pallas_call(
        _ln_body_chain,
        grid=(nblk,),
        in_specs=[pl.BlockSpec(memory_space=pl.ANY)] + data_specs,
        out_specs=out_spec,
        out_shape=out_shape,
        input_output_aliases={0: 0},
    )(prev, ex_c, segf, ep, segw, gamma2, beta2)


def kernel(x, seg, emb_x_w, emb_pos_w, emb_seg_w, gamma, beta):
    b, l = x.shape
    e = emb_x_w.shape[1]
    bc = b // _NCHUNK
    nblk = bc // _BB
    bf = jnp.bfloat16
    xi = x.astype(jnp.int32)
    segf = seg.astype(bf)
    ep = emb_pos_w[:l].astype(bf)
    segw = emb_seg_w.astype(bf)
    gamma2 = gamma.astype(bf).reshape(1, e)
    beta2 = beta.astype(bf).reshape(1, e)

    exs = [
        _sc_gather(emb_x_w, xi[c * bc:(c + 1) * bc].reshape(1, bc * l))
        .reshape(bc, l, e)
        for c in range(_NCHUNK)
    ]
    out = None
    for c in range(_NCHUNK):
        out = _tc_chunk(out, exs[c], segf, ep, segw, gamma2, beta2, c, nblk)
    return out


# speedup vs baseline: 1.0459x; 1.0459x over previous
import functools

import jax
import jax.numpy as jnp
from jax import lax
from jax.experimental import pallas as pl
from jax.experimental.pallas import tpu as pltpu
from jax.experimental.pallas import tpu_sc as plsc

_W = 128
_NCHUNK = 8
_BB = 8


def _sc_gather(table, idx):
    n = idx.shape[1]
    e = table.shape[1]
    mesh = plsc.VectorSubcoreMesh(core_axis_name="core", subcore_axis_name="subcore")

    @functools.partial(
        pl.kernel,
        out_type=jax.ShapeDtypeStruct((n, e), table.dtype),
        mesh=mesh,
    )
    def gather_kernel(table_hbm, idx_hbm, out_hbm):
        def body(idx_vmem, out_vmem):
            pltpu.sync_copy(table_hbm.at[idx_vmem.at[0]], out_vmem)

        pltpu.emit_pipeline(
            body,
            grid=(n // _W,),
            in_specs=[pl.BlockSpec((1, _W), index_map=lambda i: (0, i))],
            out_specs=[pl.BlockSpec((_W, e), index_map=lambda i: (i, 0))],
            core_axis_name=("core", "subcore"),
            dimension_semantics=(pltpu.PARALLEL,),
        )(idx_hbm, out_hbm)

    return gather_kernel(table, idx)


def _ln_math(ex_ref, segf_ref, ep_ref, segw_ref, gamma_ref, beta_ref, out_ref):
    ex = ex_ref[...].astype(jnp.bfloat16)
    segf = segf_ref[...]
    ep = ep_ref[...] + segw_ref[0]
    ds = segw_ref[1] - segw_ref[0]
    h = ex + ep[None, :, :] + segf[:, :, None] * ds[None, None, :]
    mean = jnp.mean(h, axis=-1, keepdims=True)
    d = h - mean
    var = jnp.mean(d * d, axis=-1, keepdims=True)
    inv = lax.rsqrt(var.astype(jnp.float32) + 1e-5).astype(jnp.bfloat16)
    y = d * inv * gamma_ref[0][None, None, :] + beta_ref[0][None, None, :]
    out_ref[...] = y.astype(jnp.float32)


def _ln_body_first(ex_ref, segf_ref, ep_ref, segw_ref, gamma_ref, beta_ref, out_ref):
    _ln_math(ex_ref, segf_ref, ep_ref, segw_ref, gamma_ref, beta_ref, out_ref)


def _ln_body_chain(prev_ref, ex_ref, segf_ref, ep_ref, segw_ref, gamma_ref,
                   beta_ref, out_ref):
    del prev_ref
    _ln_math(ex_ref, segf_ref, ep_ref, segw_ref, gamma_ref, beta_ref, out_ref)


def _tc_chunk(prev, ex_c, segf, ep, segw, gamma2, beta2, c, nblk):
    b, l = segf.shape
    e = ex_c.shape[-1]
    off = c * nblk
    data_specs = [
        pl.BlockSpec((_BB, l, e), lambda i: (i, 0, 0)),
        pl.BlockSpec((_BB, l), lambda i, off=off: (off + i, 0)),
        pl.BlockSpec((l, e), lambda i: (0, 0)),
        pl.BlockSpec((2, e), lambda i: (0, 0)),
        pl.BlockSpec((1, e), lambda i: (0, 0)),
        pl.BlockSpec((1, e), lambda i: (0, 0)),
    ]
    out_spec = pl.BlockSpec((_BB, l, e), lambda i, off=off: (off + i, 0, 0))
    out_shape = jax.ShapeDtypeStruct((b, l, e), jnp.float32)
    if prev is None:
        return pl.pallas_call(
            _ln_body_first,
            grid=(nblk,),
            in_specs=data_specs,
            out_specs=out_spec,
            out_shape=out_shape,
        )(ex_c, segf, ep, segw, gamma2, beta2)
    return pl.pallas_call(
        _ln_body_chain,
        grid=(nblk,),
        in_specs=[pl.BlockSpec(memory_space=pl.ANY)] + data_specs,
        out_specs=out_spec,
        out_shape=out_shape,
        input_output_aliases={0: 0},
    )(prev, ex_c, segf, ep, segw, gamma2, beta2)


def kernel(x, seg, emb_x_w, emb_pos_w, emb_seg_w, gamma, beta):
    b, l = x.shape
    e = emb_x_w.shape[1]
    bc = b // _NCHUNK
    nblk = bc // _BB
    bf = jnp.bfloat16
    xi = x.astype(jnp.int32)
    segf = seg.astype(bf)
    ep = emb_pos_w[:l].astype(bf)
    segw = emb_seg_w.astype(bf)
    gamma2 = gamma.astype(bf).reshape(1, e)
    beta2 = beta.astype(bf).reshape(1, e)

    exs = [
        _sc_gather(emb_x_w, xi[c * bc:(c + 1) * bc].reshape(1, bc * l))
        .reshape(bc, l, e)
        for c in range(_NCHUNK)
    ]
    out = None
    for c in range(_NCHUNK):
        out = _tc_chunk(out, exs[c], segf, ep, segw, gamma2, beta2, c, nblk)
    return out


# speedup vs baseline: 1.0520x; 1.0058x over previous
import functools

import jax
import jax.numpy as jnp
from jax import lax
from jax.experimental import pallas as pl
from jax.experimental.pallas import tpu as pltpu
from jax.experimental.pallas import tpu_sc as plsc

_W = 128
_NCHUNK = 4
_BB = 8


def _sc_gather(table, idx):
    n = idx.shape[1]
    e = table.shape[1]
    mesh = plsc.VectorSubcoreMesh(core_axis_name="core", subcore_axis_name="subcore")

    @functools.partial(
        pl.kernel,
        out_type=jax.ShapeDtypeStruct((n, e), table.dtype),
        mesh=mesh,
    )
    def gather_kernel(table_hbm, idx_hbm, out_hbm):
        def body(idx_vmem, out_vmem):
            pltpu.sync_copy(table_hbm.at[idx_vmem.at[0]], out_vmem)

        pltpu.emit_pipeline(
            body,
            grid=(n // _W,),
            in_specs=[pl.BlockSpec((1, _W), index_map=lambda i: (0, i))],
            out_specs=[pl.BlockSpec((_W, e), index_map=lambda i: (i, 0))],
            core_axis_name=("core", "subcore"),
            dimension_semantics=(pltpu.PARALLEL,),
        )(idx_hbm, out_hbm)

    return gather_kernel(table, idx)


def _ln_math(ex_ref, segf_ref, ep_ref, segw_ref, gamma_ref, beta_ref, out_ref):
    ex = ex_ref[...].astype(jnp.bfloat16)
    segf = segf_ref[...]
    ep = ep_ref[...] + segw_ref[0]
    ds = segw_ref[1] - segw_ref[0]
    h = ex + ep[None, :, :] + segf[:, :, None] * ds[None, None, :]
    mean = jnp.mean(h, axis=-1, keepdims=True)
    d = h - mean
    var = jnp.mean(d * d, axis=-1, keepdims=True)
    inv = lax.rsqrt(var.astype(jnp.float32) + 1e-5).astype(jnp.bfloat16)
    y = d * inv * gamma_ref[0][None, None, :] + beta_ref[0][None, None, :]
    out_ref[...] = y.astype(jnp.float32)


def _ln_body_first(ex_ref, segf_ref, ep_ref, segw_ref, gamma_ref, beta_ref, out_ref):
    _ln_math(ex_ref, segf_ref, ep_ref, segw_ref, gamma_ref, beta_ref, out_ref)


def _ln_body_chain(prev_ref, ex_ref, segf_ref, ep_ref, segw_ref, gamma_ref,
                   beta_ref, out_ref):
    del prev_ref
    _ln_math(ex_ref, segf_ref, ep_ref, segw_ref, gamma_ref, beta_ref, out_ref)


def _tc_chunk(prev, ex_c, segf, ep, segw, gamma2, beta2, c, nblk):
    b, l = segf.shape
    e = ex_c.shape[-1]
    off = c * nblk
    data_specs = [
        pl.BlockSpec((_BB, l, e), lambda i: (i, 0, 0)),
        pl.BlockSpec((_BB, l), lambda i, off=off: (off + i, 0)),
        pl.BlockSpec((l, e), lambda i: (0, 0)),
        pl.BlockSpec((2, e), lambda i: (0, 0)),
        pl.BlockSpec((1, e), lambda i: (0, 0)),
        pl.BlockSpec((1, e), lambda i: (0, 0)),
    ]
    out_spec = pl.BlockSpec((_BB, l, e), lambda i, off=off: (off + i, 0, 0))
    out_shape = jax.ShapeDtypeStruct((b, l, e), jnp.float32)
    cp = pltpu.CompilerParams(skip_device_barrier=True)
    if prev is None:
        return pl.pallas_call(
            _ln_body_first,
            grid=(nblk,),
            in_specs=data_specs,
            out_specs=out_spec,
            out_shape=out_shape,
            compiler_params=cp,
        )(ex_c, segf, ep, segw, gamma2, beta2)
    return pl.pallas_call(
        _ln_body_chain,
        grid=(nblk,),
        in_specs=[pl.BlockSpec(memory_space=pl.ANY)] + data_specs,
        out_specs=out_spec,
        out_shape=out_shape,
        input_output_aliases={0: 0},
        compiler_params=cp,
    )(prev, ex_c, segf, ep, segw, gamma2, beta2)


def kernel(x, seg, emb_x_w, emb_pos_w, emb_seg_w, gamma, beta):
    b, l = x.shape
    e = emb_x_w.shape[1]
    bc = b // _NCHUNK
    nblk = bc // _BB
    bf = jnp.bfloat16
    xi = x.astype(jnp.int32)
    segf = seg.astype(bf)
    ep = emb_pos_w[:l].astype(bf)
    segw = emb_seg_w.astype(bf)
    gamma2 = gamma.astype(bf).reshape(1, e)
    beta2 = beta.astype(bf).reshape(1, e)

    exs = [
        _sc_gather(emb_x_w, xi[c * bc:(c + 1) * bc].reshape(1, bc * l))
        .reshape(bc, l, e)
        for c in range(_NCHUNK)
    ]
    out = None
    for c in range(_NCHUNK):
        out = _tc_chunk(out, exs[c], segf, ep, segw, gamma2, beta2, c, nblk)
    return out


# speedup vs baseline: 1.2058x; 1.1462x over previous
import functools

import jax
import jax.numpy as jnp
from jax import lax
from jax.experimental import pallas as pl
from jax.experimental.pallas import tpu as pltpu
from jax.experimental.pallas import tpu_sc as plsc

_W = 128
_NCHUNK = 4
_BB = 16


def _sc_gather(table, idx, n, base):
    e = table.shape[1]
    w0 = base // _W
    mesh = plsc.VectorSubcoreMesh(core_axis_name="core", subcore_axis_name="subcore")

    @functools.partial(
        pl.kernel,
        out_type=jax.ShapeDtypeStruct((n, e), table.dtype),
        mesh=mesh,
    )
    def gather_kernel(table_hbm, idx_hbm, out_hbm):
        def body(idx_vmem, out_vmem):
            pltpu.sync_copy(table_hbm.at[idx_vmem.at[0]], out_vmem)

        pltpu.emit_pipeline(
            body,
            grid=(n // _W,),
            in_specs=[pl.BlockSpec((1, _W), index_map=lambda i: (0, w0 + i))],
            out_specs=[pl.BlockSpec((_W, e), index_map=lambda i: (i, 0))],
            core_axis_name=("core", "subcore"),
            dimension_semantics=(pltpu.PARALLEL,),
        )(idx_hbm, out_hbm)

    return gather_kernel(table, idx)


def _ln_math(ex_ref, segf_ref, ep_ref, segw_ref, gamma_ref, beta_ref, out_ref):
    ex = ex_ref[...].astype(jnp.bfloat16)
    segf = segf_ref[...]
    ep = ep_ref[...] + segw_ref[0]
    ds = segw_ref[1] - segw_ref[0]
    h = ex + ep[None, :, :] + segf[:, :, None] * ds[None, None, :]
    mean = jnp.mean(h, axis=-1, keepdims=True)
    d = h - mean
    var = jnp.mean(d * d, axis=-1, keepdims=True)
    inv = lax.rsqrt(var.astype(jnp.float32) + 1e-5).astype(jnp.bfloat16)
    y = d * inv * gamma_ref[0][None, None, :] + beta_ref[0][None, None, :]
    out_ref[...] = y.astype(jnp.float32)


def _ln_body_first(ex_ref, segf_ref, ep_ref, segw_ref, gamma_ref, beta_ref, out_ref):
    _ln_math(ex_ref, segf_ref, ep_ref, segw_ref, gamma_ref, beta_ref, out_ref)


def _ln_body_chain(prev_ref, ex_ref, segf_ref, ep_ref, segw_ref, gamma_ref,
                   beta_ref, out_ref):
    del prev_ref
    _ln_math(ex_ref, segf_ref, ep_ref, segw_ref, gamma_ref, beta_ref, out_ref)


def _tc_chunk(prev, ex_c, segf, ep, segw, gamma2, beta2, c, nblk):
    b, l = segf.shape
    e = ex_c.shape[-1]
    off = c * nblk
    data_specs = [
        pl.BlockSpec((_BB, l, e), lambda i: (i, 0, 0)),
        pl.BlockSpec((_BB, l), lambda i, off=off: (off + i, 0)),
        pl.BlockSpec((l, e), lambda i: (0, 0)),
        pl.BlockSpec((2, e), lambda i: (0, 0)),
        pl.BlockSpec((1, e), lambda i: (0, 0)),
        pl.BlockSpec((1, e), lambda i: (0, 0)),
    ]
    out_spec = pl.BlockSpec((_BB, l, e), lambda i, off=off: (off + i, 0, 0))
    out_shape = jax.ShapeDtypeStruct((b, l, e), jnp.float32)
    cp = pltpu.CompilerParams(skip_device_barrier=True)
    if prev is None:
        return pl.pallas_call(
            _ln_body_first,
            grid=(nblk,),
            in_specs=data_specs,
            out_specs=out_spec,
            out_shape=out_shape,
            compiler_params=cp,
        )(ex_c, segf, ep, segw, gamma2, beta2)
    return pl.pallas_call(
        _ln_body_chain,
        grid=(nblk,),
        in_specs=[pl.BlockSpec(memory_space=pl.ANY)] + data_specs,
        out_specs=out_spec,
        out_shape=out_shape,
        input_output_aliases={0: 0},
        compiler_params=cp,
    )(prev, ex_c, segf, ep, segw, gamma2, beta2)


def kernel(x, seg, emb_x_w, emb_pos_w, emb_seg_w, gamma, beta):
    b, l = x.shape
    e = emb_x_w.shape[1]
    bc = b // _NCHUNK
    nblk = bc // _BB
    bf = jnp.bfloat16
    xi = x.astype(jnp.int32)
    segf = seg.astype(bf)
    ep = emb_pos_w[:l].astype(bf)
    segw = emb_seg_w.astype(bf)
    gamma2 = gamma.astype(bf).reshape(1, e)
    beta2 = beta.astype(bf).reshape(1, e)

    idx_full = xi.reshape(1, b * l)
    exs = [
        _sc_gather(emb_x_w, idx_full, bc * l, c * bc * l).reshape(bc, l, e)
        for c in range(_NCHUNK)
    ]
    out = None
    for c in range(_NCHUNK):
        out = _tc_chunk(out, exs[c], segf, ep, segw, gamma2, beta2, c, nblk)
    return out


# speedup vs baseline: 1.2204x; 1.0121x over previous
import functools

import jax
import jax.numpy as jnp
from jax import lax
from jax.experimental import pallas as pl
from jax.experimental.pallas import tpu as pltpu
from jax.experimental.pallas import tpu_sc as plsc

_W = 128
_NCHUNK = 4
_BB = 32


def _sc_gather(table, idx, n, base):
    e = table.shape[1]
    w0 = base // _W
    mesh = plsc.VectorSubcoreMesh(core_axis_name="core", subcore_axis_name="subcore")

    @functools.partial(
        pl.kernel,
        out_type=jax.ShapeDtypeStruct((n, e), table.dtype),
        mesh=mesh,
    )
    def gather_kernel(table_hbm, idx_hbm, out_hbm):
        def body(idx_vmem, out_vmem):
            pltpu.sync_copy(table_hbm.at[idx_vmem.at[0]], out_vmem)

        pltpu.emit_pipeline(
            body,
            grid=(n // _W,),
            in_specs=[pl.BlockSpec((1, _W), index_map=lambda i: (0, w0 + i))],
            out_specs=[pl.BlockSpec((_W, e), index_map=lambda i: (i, 0))],
            core_axis_name=("core", "subcore"),
            dimension_semantics=(pltpu.PARALLEL,),
        )(idx_hbm, out_hbm)

    return gather_kernel(table, idx)


def _ln_math(ex_ref, segf_ref, ep_ref, segw_ref, gamma_ref, beta_ref, out_ref):
    ex = ex_ref[...].astype(jnp.bfloat16)
    segf = segf_ref[...]
    ep = ep_ref[...] + segw_ref[0]
    ds = segw_ref[1] - segw_ref[0]
    h = ex + ep[None, :, :] + segf[:, :, None] * ds[None, None, :]
    mean = jnp.mean(h, axis=-1, keepdims=True)
    d = h - mean
    var = jnp.mean(d * d, axis=-1, keepdims=True)
    inv = lax.rsqrt(var.astype(jnp.float32) + 1e-5).astype(jnp.bfloat16)
    y = d * inv * gamma_ref[0][None, None, :] + beta_ref[0][None, None, :]
    out_ref[...] = y.astype(jnp.float32)


def _ln_body_first(ex_ref, segf_ref, ep_ref, segw_ref, gamma_ref, beta_ref, out_ref):
    _ln_math(ex_ref, segf_ref, ep_ref, segw_ref, gamma_ref, beta_ref, out_ref)


def _ln_body_chain(prev_ref, ex_ref, segf_ref, ep_ref, segw_ref, gamma_ref,
                   beta_ref, out_ref):
    del prev_ref
    _ln_math(ex_ref, segf_ref, ep_ref, segw_ref, gamma_ref, beta_ref, out_ref)


def _tc_chunk(prev, ex_c, segf, ep, segw, gamma2, beta2, c, nblk):
    b, l = segf.shape
    e = ex_c.shape[-1]
    off = c * nblk
    data_specs = [
        pl.BlockSpec((_BB, l, e), lambda i: (i, 0, 0)),
        pl.BlockSpec((_BB, l), lambda i, off=off: (off + i, 0)),
        pl.BlockSpec((l, e), lambda i: (0, 0)),
        pl.BlockSpec((2, e), lambda i: (0, 0)),
        pl.BlockSpec((1, e), lambda i: (0, 0)),
        pl.BlockSpec((1, e), lambda i: (0, 0)),
    ]
    out_spec = pl.BlockSpec((_BB, l, e), lambda i, off=off: (off + i, 0, 0))
    out_shape = jax.ShapeDtypeStruct((b, l, e), jnp.float32)
    cp = pltpu.CompilerParams(skip_device_barrier=True)
    if prev is None:
        return pl.pallas_call(
            _ln_body_first,
            grid=(nblk,),
            in_specs=data_specs,
            out_specs=out_spec,
            out_shape=out_shape,
            compiler_params=cp,
        )(ex_c, segf, ep, segw, gamma2, beta2)
    return pl.pallas_call(
        _ln_body_chain,
        grid=(nblk,),
        in_specs=[pl.BlockSpec(memory_space=pl.ANY)] + data_specs,
        out_specs=out_spec,
        out_shape=out_shape,
        input_output_aliases={0: 0},
        compiler_params=cp,
    )(prev, ex_c, segf, ep, segw, gamma2, beta2)


def kernel(x, seg, emb_x_w, emb_pos_w, emb_seg_w, gamma, beta):
    b, l = x.shape
    e = emb_x_w.shape[1]
    bc = b // _NCHUNK
    nblk = bc // _BB
    bf = jnp.bfloat16
    xi = x.astype(jnp.int32)
    segf = seg.astype(bf)
    ep = emb_pos_w[:l].astype(bf)
    segw = emb_seg_w.astype(bf)
    gamma2 = gamma.astype(bf).reshape(1, e)
    beta2 = beta.astype(bf).reshape(1, e)

    idx_full = xi.reshape(1, b * l)
    exs = [
        _sc_gather(emb_x_w, idx_full, bc * l, c * bc * l).reshape(bc, l, e)
        for c in range(_NCHUNK)
    ]
    out = None
    for c in range(_NCHUNK):
        out = _tc_chunk(out, exs[c], segf, ep, segw, gamma2, beta2, c, nblk)
    return out
